# baseline (speedup 1.0000x reference)
import dataclasses
import functools

import jax
import jax.numpy as jnp
from jax import lax
from jax.experimental import pallas as pl
from jax.experimental.pallas import tpu as pltpu
from jax.experimental.pallas import tpu_sc as plsc

_N = 10000
_E = 320000
_D_IN = 128
_HID = 64
_HEADS = 8
_B = 32
_CONCAT = [True, True, False]
_HD = _HEADS * _HID
_ACW = _HD + 16

_NWORK = 32
_NPASS = 2
_NGROUP = _NWORK * _NPASS
_CH = 160
_ACC_ROWS = _CH + 1
_NTILE = 16
_EBLK = 2000
_NBLK = _E // _EBLK
_XR_PAD_ROWS = _NGROUP * _CH + 16
_VL = 16

_PREC = lax.Precision.HIGHEST
_DEBUG_JNP_EDGE = False


def _dot(a, b):
    return lax.dot_general(a, b, (((1,), (0,)), ((), ())),
                           preferred_element_type=jnp.float32,
                           precision=_PREC)


def _dot_bf(a, b):
    return lax.dot_general(a.astype(jnp.bfloat16), b.astype(jnp.bfloat16),
                           (((1,), (0,)), ((), ())),
                           preferred_element_type=jnp.float32)


def _proj_body(x_ref, w_ref, b_ref, o_ref):
    o_ref[...] = jnp.maximum(_dot_bf(x_ref[...], w_ref[...]) + b_ref[...],
                             0.0)


def _proj(x, w, b):
    blk = 2000
    return pl.pallas_call(
        _proj_body,
        grid=(_N // blk,),
        in_specs=[
            pl.BlockSpec((blk, _D_IN), lambda i: (i, 0)),
            pl.BlockSpec((_D_IN, _HID), lambda i: (0, 0)),
            pl.BlockSpec((1, _HID), lambda i: (0, 0)),
        ],
        out_specs=pl.BlockSpec((blk, _HID), lambda i: (i, 0)),
        out_shape=jax.ShapeDtypeStruct((_N, _HID), jnp.float32),
    )(x, w, b)


def _mm2_body(h_ref, wl_ref, bl_ref, wr_ref, br_ref, xl_ref, xr_ref):
    h = h_ref[...]
    xl_ref[...] = _dot_bf(h, wl_ref[...]) + bl_ref[...]
    xr_ref[...] = _dot_bf(h, wr_ref[...]) + br_ref[...]


def _mm2(h, wl, bl, wr, br):
    blk = 1000
    d_in = h.shape[1]
    return pl.pallas_call(
        _mm2_body,
        grid=(_N // blk,),
        in_specs=[
            pl.BlockSpec((blk, d_in), lambda i: (i, 0)),
            pl.BlockSpec((d_in, _HD), lambda i: (0, 0)),
            pl.BlockSpec((1, _HD), lambda i: (0, 0)),
            pl.BlockSpec((d_in, _HD), lambda i: (0, 0)),
            pl.BlockSpec((1, _HD), lambda i: (0, 0)),
        ],
        out_specs=[
            pl.BlockSpec((blk, _HD), lambda i: (i, 0)),
            pl.BlockSpec((blk, _HD), lambda i: (i, 0)),
        ],
        out_shape=[
            jax.ShapeDtypeStruct((_N, _HD), jnp.float32),
            jax.ShapeDtypeStruct((_N, _HD), jnp.float32),
        ],
    )(h, wl, bl, wr, br)


def _self_body(xl_ref, xr_ref, a_ref, om_ref, od_ref):
    xl = xl_ref[...]
    z = xl + xr_ref[...]
    t = jnp.maximum(z, 0.2 * z)
    ex = jnp.exp(_dot(t, a_ref[...]))
    for h in range(_HEADS):
        om_ref[:, h * _HID:(h + 1) * _HID] = (
            xl[:, h * _HID:(h + 1) * _HID] * ex[:, h:h + 1])
    od_ref[...] = ex


def _selfloop(xl, xr, a_blockdiag):
    blk = 1000
    return pl.pallas_call(
        _self_body,
        grid=(_N // blk,),
        in_specs=[
            pl.BlockSpec((blk, _HD), lambda i: (i, 0)),
            pl.BlockSpec((blk, _HD), lambda i: (i, 0)),
            pl.BlockSpec((_HD, _HEADS), lambda i: (0, 0)),
        ],
        out_specs=[
            pl.BlockSpec((blk, _HD), lambda i: (i, 0)),
            pl.BlockSpec((blk, _HEADS), lambda i: (i, 0)),
        ],
        out_shape=[
            jax.ShapeDtypeStruct((_N, _HD), jnp.float32),
            jax.ShapeDtypeStruct((_N, _HEADS), jnp.float32),
        ],
    )(xl, xr, a_blockdiag)


def _epi_a_body(scm_ref, scd_ref, sem_ref, sed_ref, bias_ref,
                o_ref, *, concat):
    scm = scm_ref[...]
    sem = sem_ref[...]
    den = scd_ref[...][:, :_HEADS] + sed_ref[...]
    inv = 1.0 / (den + 1e-16)
    cols = []
    for h in range(_HEADS):
        cols.append((scm[:, h * _HID:(h + 1) * _HID]
                     + sem[:, h * _HID:(h + 1) * _HID]) * inv[:, h:h + 1])
    if concat:
        o = jnp.concatenate(cols, axis=1) + bias_ref[...]
    else:
        acc = cols[0]
        for c in cols[1:]:
            acc = acc + c
        o = acc * (1.0 / _HEADS) + bias_ref[...]
    o_ref[...] = o


def _epi_a(scm, scd, sem, sed, bias, concat):
    blk = 1000
    nblk = _N // blk
    c = _HD if concat else _HID
    return pl.pallas_call(
        functools.partial(_epi_a_body, concat=concat),
        grid=(nblk,),
        in_specs=[
            pl.BlockSpec((blk, _HD), lambda i: (i, 0)),
            pl.BlockSpec((blk, 16), lambda i: (i, 0)),
            pl.BlockSpec((blk, _HD), lambda i: (i, 0)),
            pl.BlockSpec((blk, _HEADS), lambda i: (i, 0)),
            pl.BlockSpec((1, c), lambda i: (0, 0)),
        ],
        out_specs=pl.BlockSpec((blk, c), lambda i: (i, 0)),
        out_shape=jax.ShapeDtypeStruct((_N, c), jnp.float32),
    )(scm, scd, sem, sed, bias)


def _colsum(o, s=None, center=False):
    blk = 1000
    nblk = _N // blk
    c = o.shape[1]

    def body(*refs):
        if center:
            o_ref, s_ref, out_ref, acc_ref = refs
        else:
            o_ref, out_ref, acc_ref = refs
            s_ref = None
        i = pl.program_id(0)

        @pl.when(i == 0)
        def _():
            acc_ref[...] = jnp.zeros_like(acc_ref)

        o = o_ref[...]
        if center:
            d = o - s_ref[...] * (1.0 / _N)
            acc_ref[...] += jnp.sum(d * d, axis=0, keepdims=True)
        else:
            acc_ref[...] += jnp.sum(o, axis=0, keepdims=True)

        @pl.when(i == nblk - 1)
        def _():
            out_ref[...] = acc_ref[...]

    in_specs = [pl.BlockSpec((blk, c), lambda i: (i, 0))]
    args = [o]
    if center:
        in_specs.append(pl.BlockSpec((1, c), lambda i: (0, 0)))
        args.append(s)
    return pl.pallas_call(
        body,
        grid=(nblk,),
        in_specs=in_specs,
        out_specs=pl.BlockSpec((1, c), lambda i: (0, 0)),
        out_shape=jax.ShapeDtypeStruct((1, c), jnp.float32),
        scratch_shapes=[pltpu.VMEM((1, c), jnp.float32)],
    )(*args)


def _bn_norm_body(o_ref, s1_ref, vs_ref, g_ref, b_ref, out_ref, *, do_relu):
    m = s1_ref[...] * (1.0 / _N)
    v = vs_ref[...] * (1.0 / _N)
    hh = (o_ref[...] - m) / jnp.sqrt(v + 1e-5) * g_ref[...] + b_ref[...]
    if do_relu:
        hh = jnp.maximum(hh, 0.0)
    out_ref[...] = hh


def _epi_b(o, g, b, do_relu):
    blk = 1000
    c = o.shape[1]
    s1 = _colsum(o)
    vs = _colsum(o, s1, center=True)
    return pl.pallas_call(
        functools.partial(_bn_norm_body, do_relu=do_relu),
        grid=(_N // blk,),
        in_specs=[
            pl.BlockSpec((blk, c), lambda i: (i, 0)),
            pl.BlockSpec((1, c), lambda i: (0, 0)),
            pl.BlockSpec((1, c), lambda i: (0, 0)),
            pl.BlockSpec((1, c), lambda i: (0, 0)),
            pl.BlockSpec((1, c), lambda i: (0, 0)),
        ],
        out_specs=pl.BlockSpec((blk, c), lambda i: (i, 0)),
        out_shape=jax.ShapeDtypeStruct((_N, c), jnp.float32),
    )(o, s1, vs, g, b)


def _head_body(h_ref, batch_ref, w1_ref, b1_ref, g_ref, bb_ref, w2_ref,
               b2_ref, o_ref):
    bidx = batch_ref[...]
    rows = lax.broadcasted_iota(jnp.int32, (_B, _N), 0)
    onehot = (bidx == rows).astype(jnp.float32)
    cnt = jnp.sum(onehot, axis=1, keepdims=True)
    pooled = _dot(onehot, h_ref[...]) / jnp.maximum(cnt, 1.0)
    z = _dot_bf(pooled, w1_ref[...]) + b1_ref[...]
    m = jnp.mean(z, axis=0, keepdims=True)
    zd = z - m
    v = jnp.mean(zd * zd, axis=0, keepdims=True)
    z = zd / jnp.sqrt(v + 1e-5) * g_ref[...] + bb_ref[...]
    z = jnp.maximum(z, 0.0)
    o_ref[...] = _dot_bf(z, w2_ref[...]) + b2_ref[...]


def _head(h, batch2d, w1, b1, g, bb, w2, b2):
    return pl.pallas_call(
        _head_body,
        out_shape=jax.ShapeDtypeStruct((_B, 1), jnp.float32),
    )(h, batch2d, w1, b1, g, bb, w2, b2)


def _sc_edge(xl, xr_pad, srcv, dstv, attb, zrows):
    mesh = plsc.VectorSubcoreMesh(core_axis_name="c", subcore_axis_name="s")
    cp = pltpu.CompilerParams()
    if "needs_layout_passes" in pltpu.CompilerParams.__dataclass_fields__:
        cp = dataclasses.replace(cp, needs_layout_passes=False)

    @functools.partial(
        pl.kernel,
        compiler_params=cp,
        out_type=jax.ShapeDtypeStruct((_NGROUP, _CH, _ACW), jnp.float32),
        mesh=mesh,
        scratch_types=[
            pltpu.VMEM((_ACC_ROWS, _ACW), jnp.float32),
            pltpu.VMEM((_EBLK,), jnp.int32),
            pltpu.VMEM((_EBLK,), jnp.int32),
            pltpu.VMEM((32,), jnp.int32),
            pltpu.VMEM((32,), jnp.int32),
            pltpu.VMEM((_VL, _HD), jnp.float32),
            pltpu.VMEM((_VL, _HD), jnp.float32),
            pltpu.VMEM((_HD,), jnp.float32),
        ],
    )
    def kern(xl_hbm, xr_hbm, src_hbm, dst_hbm, attb_hbm, z_hbm, out_hbm,
             acc, dbuf, sbuf, std, sts, xl16, xr16, attv):
        cid = lax.axis_index("c")
        sid = lax.axis_index("s")
        wid = cid * 16 + sid
        li = lax.iota(jnp.int32, _VL)

        pltpu.sync_copy(attb_hbm, attv)

        def drain16(d_vec, s_vec, lo):
            pltpu.sync_copy(xl_hbm.at[s_vec], xl16)
            pltpu.sync_copy(xr_hbm.at[d_vec], xr16)
            dloc = d_vec - jnp.full((_VL,), lo, jnp.int32)
            for h in range(_HEADS):
                def abody(d4, a_v, _h=h):
                    base = d4 * 4 + _h * _HID
                    for k in range(4):
                        colv = jnp.full((_VL,), base + k, jnp.int32)
                        xa = plsc.load_gather(xl16, [li, colv])
                        xb = plsc.load_gather(xr16, [li, colv])
                        z = xa + xb
                        t = jnp.maximum(z, 0.2 * z)
                        av = plsc.load_gather(attv, [colv])
                        a_v = a_v + t * av
                    return a_v

                alpha = lax.fori_loop(0, _HID // 4, abody,
                                      jnp.zeros((_VL,), jnp.float32))
                ex = jnp.exp(alpha)
                plsc.addupdate_scatter(
                    acc, [dloc, jnp.full((_VL,), _HD + h, jnp.int32)], ex)

                def wbody(d4, carry, _h=h, _ex=ex):
                    base = d4 * 4 + _h * _HID
                    for k in range(4):
                        colv = jnp.full((_VL,), base + k, jnp.int32)
                        xa = plsc.load_gather(xl16, [li, colv])
                        plsc.addupdate_scatter(acc, [dloc, colv], xa * _ex)
                    return carry

                lax.fori_loop(0, _HID // 4, wbody, 0)

        for p in range(_NPASS):
            g = p * _NWORK + wid
            lo = g * _CH
            hi = lo + _CH

            pltpu.sync_copy(z_hbm, acc)

            def blk_body(bi, off, lo=lo, hi=hi):
                pltpu.sync_copy(dst_hbm.at[pl.ds(bi * _EBLK, _EBLK)],
                                dbuf)
                pltpu.sync_copy(src_hbm.at[pl.ds(bi * _EBLK, _EBLK)],
                                sbuf)

                def j_body(j, off):
                    d = dbuf[pl.ds(j * _VL, _VL)]
                    s = sbuf[pl.ds(j * _VL, _VL)]
                    lov = jnp.full((_VL,), lo, jnp.int32)
                    hiv = jnp.full((_VL,), hi, jnp.int32)
                    m = (d >= lov) & (d < hiv) & (d != s)
                    cnt = jnp.sum(m.astype(jnp.int32), axis=0)

                    @pl.when(cnt > 0)
                    def _():
                        plsc.store_compressed(std.at[pl.ds(off, _VL)], d,
                                              mask=m)
                        plsc.store_compressed(sts.at[pl.ds(off, _VL)], s,
                                              mask=m)

                    off = off + cnt

                    def do_drain(o):
                        drain16(std[pl.ds(0, _VL)], sts[pl.ds(0, _VL)], lo)
                        d2 = std[pl.ds(_VL, _VL)]
                        s2 = sts[pl.ds(_VL, _VL)]
                        std[pl.ds(0, _VL)] = d2
                        sts[pl.ds(0, _VL)] = s2
                        return o - _VL

                    return lax.cond(off >= _VL, do_drain, lambda o: o, off)

                return lax.fori_loop(0, _EBLK // _VL, j_body, off)

            off = lax.fori_loop(0, _NBLK, blk_body, jnp.int32(0))

            @pl.when(off > 0)
            def _(off=off, lo=lo):
                offv = jnp.full((_VL,), off, jnp.int32)
                dumpv = jnp.full((_VL,), lo + _CH, jnp.int32)
                dv = jnp.where(li < offv, std[pl.ds(0, _VL)], dumpv)
                sv = jnp.where(li < offv, sts[pl.ds(0, _VL)],
                               jnp.zeros((_VL,), jnp.int32))
                drain16(dv, sv, lo)

            pltpu.sync_copy(acc.at[pl.ds(0, _CH)], out_hbm.at[g])

    return kern(xl, xr_pad, srcv, dstv, attb, zrows)


def kernel(x, edge_index, batch, params):
    p = params
    srcv = edge_index[0]
    dstv = edge_index[1]
    batch2d = batch.reshape(1, _N)
    zrows = jnp.zeros((_ACC_ROWS, _ACW), jnp.float32)

    h = _proj(x, p["proj_W"], p["proj_b"].reshape(1, _HID))
    for i in range(3):
        att = p[f"att{i}"]
        attb = att.reshape(_HD)
        a_bd = jnp.zeros((_HD, _HEADS), jnp.float32)
        for hh in range(_HEADS):
            a_bd = a_bd.at[hh * _HID:(hh + 1) * _HID, hh].set(att[hh])

        xl, xr = _mm2(h, p[f"Wl{i}"], p[f"bl{i}"].reshape(1, _HD),
                      p[f"Wr{i}"], p[f"br{i}"].reshape(1, _HD))
        xr_pad = jnp.concatenate(
            [xr, jnp.zeros((_XR_PAD_ROWS - _N, _HD), jnp.float32)], axis=0)

        if _DEBUG_JNP_EDGE:
            keep = srcv != dstv
            d0 = jnp.where(keep, dstv, _N)
            zse = xl[srcv] + xr[jnp.minimum(d0, _N - 1)]
            tse = jnp.maximum(zse, 0.2 * zse)
            al = (tse.reshape(_E, _HEADS, _HID)
                  * att[None, :, :]).sum(-1)
            exe = jnp.exp(al)
            scd8 = jax.ops.segment_sum(exe, d0, num_segments=_N)
            scm = jnp.concatenate([
                jax.ops.segment_sum(
                    xl[srcv, hh * _HID:(hh + 1) * _HID] * exe[:, hh:hh + 1],
                    d0, num_segments=_N)
                for hh in range(_HEADS)], axis=1)
            scd = jnp.concatenate(
                [scd8, jnp.zeros((_N, 8), jnp.float32)], axis=1)
        else:
            scg = _sc_edge(xl, xr_pad, srcv, dstv, attb, zrows)
            scflat = scg.reshape(_NGROUP * _CH, _ACW)
            scm = scflat[:_N, :_HD]
            scd = scflat[:_N, _HD:]
        sem, sed = _selfloop(xl, xr, a_bd)

        c = _HD if _CONCAT[i] else _HID
        o = _epi_a(scm, scd, sem, sed, p[f"bias{i}"].reshape(1, c),
                   _CONCAT[i])
        h = _epi_b(o, p[f"bn{i}_g"].reshape(1, c),
                   p[f"bn{i}_b"].reshape(1, c), do_relu=(i < 2))

    return _head(h, batch2d, p["out_W1"], p["out_b1"].reshape(1, _HID // 2),
                 p["out_bn_g"].reshape(1, _HID // 2),
                 p["out_bn_b"].reshape(1, _HID // 2),
                 p["out_W2"], p["out_b2"].reshape(1, 1))


# speedup vs baseline: 1.0558x; 1.0558x over previous
import dataclasses
import functools

import jax
import jax.numpy as jnp
from jax import lax
from jax.experimental import pallas as pl
from jax.experimental.pallas import tpu as pltpu
from jax.experimental.pallas import tpu_sc as plsc

_N = 10000
_E = 320000
_D_IN = 128
_HID = 64
_HEADS = 8
_B = 32
_CONCAT = [True, True, False]
_HD = _HEADS * _HID
_ACW = _HD + 16

_NWORK = 32
_NPASS = 2
_NGROUP = _NWORK * _NPASS
_CH = 160
_ACC_ROWS = _CH + 1
_NTILE = 16
_EBLK = 1280
_NBLK = _E // _EBLK
_XR_PAD_ROWS = _NGROUP * _CH + 16
_VL = 16

_PREC = lax.Precision.HIGHEST
_DEBUG_JNP_EDGE = False


def _dot(a, b):
    return lax.dot_general(a, b, (((1,), (0,)), ((), ())),
                           preferred_element_type=jnp.float32,
                           precision=_PREC)


def _dot_bf(a, b):
    return lax.dot_general(a.astype(jnp.bfloat16), b.astype(jnp.bfloat16),
                           (((1,), (0,)), ((), ())),
                           preferred_element_type=jnp.float32)


def _proj_body(x_ref, w_ref, b_ref, o_ref):
    o_ref[...] = jnp.maximum(_dot_bf(x_ref[...], w_ref[...]) + b_ref[...],
                             0.0)


def _proj(x, w, b):
    blk = 2000
    return pl.pallas_call(
        _proj_body,
        grid=(_N // blk,),
        in_specs=[
            pl.BlockSpec((blk, _D_IN), lambda i: (i, 0)),
            pl.BlockSpec((_D_IN, _HID), lambda i: (0, 0)),
            pl.BlockSpec((1, _HID), lambda i: (0, 0)),
        ],
        out_specs=pl.BlockSpec((blk, _HID), lambda i: (i, 0)),
        out_shape=jax.ShapeDtypeStruct((_N, _HID), jnp.float32),
    )(x, w, b)


def _mm2_body(h_ref, wl_ref, bl_ref, wr_ref, br_ref, xl_ref, xr_ref):
    h = h_ref[...]
    xl_ref[...] = _dot_bf(h, wl_ref[...]) + bl_ref[...]
    xr_ref[...] = _dot_bf(h, wr_ref[...]) + br_ref[...]


def _mm2(h, wl, bl, wr, br):
    blk = 1000
    d_in = h.shape[1]
    return pl.pallas_call(
        _mm2_body,
        grid=(_N // blk,),
        in_specs=[
            pl.BlockSpec((blk, d_in), lambda i: (i, 0)),
            pl.BlockSpec((d_in, _HD), lambda i: (0, 0)),
            pl.BlockSpec((1, _HD), lambda i: (0, 0)),
            pl.BlockSpec((d_in, _HD), lambda i: (0, 0)),
            pl.BlockSpec((1, _HD), lambda i: (0, 0)),
        ],
        out_specs=[
            pl.BlockSpec((blk, _HD), lambda i: (i, 0)),
            pl.BlockSpec((blk, _HD), lambda i: (i, 0)),
        ],
        out_shape=[
            jax.ShapeDtypeStruct((_N, _HD), jnp.float32),
            jax.ShapeDtypeStruct((_N, _HD), jnp.float32),
        ],
    )(h, wl, bl, wr, br)


def _self_body(xl_ref, xr_ref, a_ref, om_ref, od_ref):
    xl = xl_ref[...]
    z = xl + xr_ref[...]
    t = jnp.maximum(z, 0.2 * z)
    ex = jnp.exp(_dot(t, a_ref[...]))
    for h in range(_HEADS):
        om_ref[:, h * _HID:(h + 1) * _HID] = (
            xl[:, h * _HID:(h + 1) * _HID] * ex[:, h:h + 1])
    od_ref[...] = ex


def _selfloop(xl, xr, a_blockdiag):
    blk = 1000
    return pl.pallas_call(
        _self_body,
        grid=(_N // blk,),
        in_specs=[
            pl.BlockSpec((blk, _HD), lambda i: (i, 0)),
            pl.BlockSpec((blk, _HD), lambda i: (i, 0)),
            pl.BlockSpec((_HD, _HEADS), lambda i: (0, 0)),
        ],
        out_specs=[
            pl.BlockSpec((blk, _HD), lambda i: (i, 0)),
            pl.BlockSpec((blk, _HEADS), lambda i: (i, 0)),
        ],
        out_shape=[
            jax.ShapeDtypeStruct((_N, _HD), jnp.float32),
            jax.ShapeDtypeStruct((_N, _HEADS), jnp.float32),
        ],
    )(xl, xr, a_blockdiag)


def _epi_a_body(scm_ref, scd_ref, sem_ref, sed_ref, bias_ref,
                o_ref, *, concat):
    scm = scm_ref[...]
    sem = sem_ref[...]
    den = scd_ref[...][:, :_HEADS] + sed_ref[...]
    inv = 1.0 / (den + 1e-16)
    cols = []
    for h in range(_HEADS):
        cols.append((scm[:, h * _HID:(h + 1) * _HID]
                     + sem[:, h * _HID:(h + 1) * _HID]) * inv[:, h:h + 1])
    if concat:
        o = jnp.concatenate(cols, axis=1) + bias_ref[...]
    else:
        acc = cols[0]
        for c in cols[1:]:
            acc = acc + c
        o = acc * (1.0 / _HEADS) + bias_ref[...]
    o_ref[...] = o


def _epi_a(scm, scd, sem, sed, bias, concat):
    blk = 1000
    nblk = _N // blk
    c = _HD if concat else _HID
    return pl.pallas_call(
        functools.partial(_epi_a_body, concat=concat),
        grid=(nblk,),
        in_specs=[
            pl.BlockSpec((blk, _HD), lambda i: (i, 0)),
            pl.BlockSpec((blk, 16), lambda i: (i, 0)),
            pl.BlockSpec((blk, _HD), lambda i: (i, 0)),
            pl.BlockSpec((blk, _HEADS), lambda i: (i, 0)),
            pl.BlockSpec((1, c), lambda i: (0, 0)),
        ],
        out_specs=pl.BlockSpec((blk, c), lambda i: (i, 0)),
        out_shape=jax.ShapeDtypeStruct((_N, c), jnp.float32),
    )(scm, scd, sem, sed, bias)


def _colsum(o, s=None, center=False):
    blk = 1000
    nblk = _N // blk
    c = o.shape[1]

    def body(*refs):
        if center:
            o_ref, s_ref, out_ref, acc_ref = refs
        else:
            o_ref, out_ref, acc_ref = refs
            s_ref = None
        i = pl.program_id(0)

        @pl.when(i == 0)
        def _():
            acc_ref[...] = jnp.zeros_like(acc_ref)

        o = o_ref[...]
        if center:
            d = o - s_ref[...] * (1.0 / _N)
            acc_ref[...] += jnp.sum(d * d, axis=0, keepdims=True)
        else:
            acc_ref[...] += jnp.sum(o, axis=0, keepdims=True)

        @pl.when(i == nblk - 1)
        def _():
            out_ref[...] = acc_ref[...]

    in_specs = [pl.BlockSpec((blk, c), lambda i: (i, 0))]
    args = [o]
    if center:
        in_specs.append(pl.BlockSpec((1, c), lambda i: (0, 0)))
        args.append(s)
    return pl.pallas_call(
        body,
        grid=(nblk,),
        in_specs=in_specs,
        out_specs=pl.BlockSpec((1, c), lambda i: (0, 0)),
        out_shape=jax.ShapeDtypeStruct((1, c), jnp.float32),
        scratch_shapes=[pltpu.VMEM((1, c), jnp.float32)],
    )(*args)


def _bn_norm_body(o_ref, s1_ref, vs_ref, g_ref, b_ref, out_ref, *, do_relu):
    m = s1_ref[...] * (1.0 / _N)
    v = vs_ref[...] * (1.0 / _N)
    hh = (o_ref[...] - m) / jnp.sqrt(v + 1e-5) * g_ref[...] + b_ref[...]
    if do_relu:
        hh = jnp.maximum(hh, 0.0)
    out_ref[...] = hh


def _epi_b(o, g, b, do_relu):
    blk = 1000
    c = o.shape[1]
    s1 = _colsum(o)
    vs = _colsum(o, s1, center=True)
    return pl.pallas_call(
        functools.partial(_bn_norm_body, do_relu=do_relu),
        grid=(_N // blk,),
        in_specs=[
            pl.BlockSpec((blk, c), lambda i: (i, 0)),
            pl.BlockSpec((1, c), lambda i: (0, 0)),
            pl.BlockSpec((1, c), lambda i: (0, 0)),
            pl.BlockSpec((1, c), lambda i: (0, 0)),
            pl.BlockSpec((1, c), lambda i: (0, 0)),
        ],
        out_specs=pl.BlockSpec((blk, c), lambda i: (i, 0)),
        out_shape=jax.ShapeDtypeStruct((_N, c), jnp.float32),
    )(o, s1, vs, g, b)


def _head_body(h_ref, batch_ref, w1_ref, b1_ref, g_ref, bb_ref, w2_ref,
               b2_ref, o_ref):
    bidx = batch_ref[...]
    rows = lax.broadcasted_iota(jnp.int32, (_B, _N), 0)
    onehot = (bidx == rows).astype(jnp.float32)
    cnt = jnp.sum(onehot, axis=1, keepdims=True)
    pooled = _dot(onehot, h_ref[...]) / jnp.maximum(cnt, 1.0)
    z = _dot_bf(pooled, w1_ref[...]) + b1_ref[...]
    m = jnp.mean(z, axis=0, keepdims=True)
    zd = z - m
    v = jnp.mean(zd * zd, axis=0, keepdims=True)
    z = zd / jnp.sqrt(v + 1e-5) * g_ref[...] + bb_ref[...]
    z = jnp.maximum(z, 0.0)
    o_ref[...] = _dot_bf(z, w2_ref[...]) + b2_ref[...]


def _head(h, batch2d, w1, b1, g, bb, w2, b2):
    return pl.pallas_call(
        _head_body,
        out_shape=jax.ShapeDtypeStruct((_B, 1), jnp.float32),
    )(h, batch2d, w1, b1, g, bb, w2, b2)


def _sc_edge(xl, xr_pad, srcv, dstv, attb, zrows):
    mesh = plsc.VectorSubcoreMesh(core_axis_name="c", subcore_axis_name="s")
    cp = pltpu.CompilerParams()
    if "needs_layout_passes" in pltpu.CompilerParams.__dataclass_fields__:
        cp = dataclasses.replace(cp, needs_layout_passes=False)

    @functools.partial(
        pl.kernel,
        compiler_params=cp,
        out_type=jax.ShapeDtypeStruct((_NGROUP, _CH, _ACW), jnp.float32),
        mesh=mesh,
        scratch_types=[
            pltpu.VMEM((_ACC_ROWS, _ACW), jnp.float32),
            pltpu.VMEM((_EBLK,), jnp.int32),
            pltpu.VMEM((_EBLK,), jnp.int32),
            pltpu.VMEM((_EBLK,), jnp.int32),
            pltpu.VMEM((_EBLK,), jnp.int32),
            pltpu.VMEM((32,), jnp.int32),
            pltpu.VMEM((32,), jnp.int32),
            pltpu.VMEM((_VL, _HD), jnp.float32),
            pltpu.VMEM((_VL, _HD), jnp.float32),
            pltpu.VMEM((_HD,), jnp.float32),
            pltpu.SemaphoreType.DMA,
            pltpu.SemaphoreType.DMA,
            pltpu.SemaphoreType.DMA,
            pltpu.SemaphoreType.DMA,
            pltpu.SemaphoreType.DMA,
            pltpu.SemaphoreType.DMA,
        ],
    )
    def kern(xl_hbm, xr_hbm, src_hbm, dst_hbm, attb_hbm, z_hbm, out_hbm,
             acc, dbufa, sbufa, dbufb, sbufb, std, sts, xl16, xr16, attv,
             semad, semas, sembd, sembs, semg1, semg2):
        cid = lax.axis_index("c")
        sid = lax.axis_index("s")
        wid = cid * 16 + sid
        li = lax.iota(jnp.int32, _VL)

        pltpu.sync_copy(attb_hbm, attv)

        def drain16(d_vec, s_vec, lo):
            ca = pltpu.async_copy(xl_hbm.at[s_vec], xl16, semg1)
            cb = pltpu.async_copy(xr_hbm.at[d_vec], xr16, semg2)
            ca.wait()
            cb.wait()
            dloc = d_vec - jnp.full((_VL,), lo, jnp.int32)
            for h in range(_HEADS):
                def abody(d4, a_v, _h=h):
                    base = d4 * 4 + _h * _HID
                    for k in range(4):
                        colv = jnp.full((_VL,), base + k, jnp.int32)
                        xa = plsc.load_gather(xl16, [li, colv])
                        xb = plsc.load_gather(xr16, [li, colv])
                        z = xa + xb
                        t = jnp.maximum(z, 0.2 * z)
                        av = plsc.load_gather(attv, [colv])
                        a_v = a_v + t * av
                    return a_v

                alpha = lax.fori_loop(0, _HID // 4, abody,
                                      jnp.zeros((_VL,), jnp.float32))
                ex = jnp.exp(alpha)
                plsc.addupdate_scatter(
                    acc, [dloc, jnp.full((_VL,), _HD + h, jnp.int32)], ex)

                def wbody(d4, carry, _h=h, _ex=ex):
                    base = d4 * 4 + _h * _HID
                    for k in range(4):
                        colv = jnp.full((_VL,), base + k, jnp.int32)
                        xa = plsc.load_gather(xl16, [li, colv])
                        plsc.addupdate_scatter(acc, [dloc, colv], xa * _ex)
                    return carry

                lax.fori_loop(0, _HID // 4, wbody, 0)

        for p in range(_NPASS):
            g = p * _NWORK + wid
            lo = g * _CH
            hi = lo + _CH

            pltpu.sync_copy(z_hbm, acc)

            def scan_block(db, sb, off, lo=lo, hi=hi):
                def j_body(j, off):
                    d = db[pl.ds(j * _VL, _VL)]
                    s = sb[pl.ds(j * _VL, _VL)]
                    lov = jnp.full((_VL,), lo, jnp.int32)
                    hiv = jnp.full((_VL,), hi, jnp.int32)
                    m = (d >= lov) & (d < hiv) & (d != s)
                    cnt = jnp.sum(m.astype(jnp.int32), axis=0)

                    @pl.when(cnt > 0)
                    def _():
                        plsc.store_compressed(std.at[pl.ds(off, _VL)], d,
                                              mask=m)
                        plsc.store_compressed(sts.at[pl.ds(off, _VL)], s,
                                              mask=m)

                    off = off + cnt

                    def do_drain(o):
                        drain16(std[pl.ds(0, _VL)], sts[pl.ds(0, _VL)], lo)
                        d2 = std[pl.ds(_VL, _VL)]
                        s2 = sts[pl.ds(_VL, _VL)]
                        std[pl.ds(0, _VL)] = d2
                        sts[pl.ds(0, _VL)] = s2
                        return o - _VL

                    return lax.cond(off >= _VL, do_drain, lambda o: o, off)

                return lax.fori_loop(0, _EBLK // _VL, j_body, off)

            pltpu.async_copy(dst_hbm.at[pl.ds(0, _EBLK)], dbufa, semad)
            pltpu.async_copy(src_hbm.at[pl.ds(0, _EBLK)], sbufa, semas)

            def blk2_body(i, off, lo=lo, hi=hi):
                b0 = 2 * i
                pltpu.make_async_copy(
                    dst_hbm.at[pl.ds(0, _EBLK)], dbufa, semad).wait()
                pltpu.make_async_copy(
                    src_hbm.at[pl.ds(0, _EBLK)], sbufa, semas).wait()
                pltpu.async_copy(
                    dst_hbm.at[pl.ds((b0 + 1) * _EBLK, _EBLK)], dbufb, sembd)
                pltpu.async_copy(
                    src_hbm.at[pl.ds((b0 + 1) * _EBLK, _EBLK)], sbufb, sembs)
                off = scan_block(dbufa, sbufa, off)
                pltpu.make_async_copy(
                    dst_hbm.at[pl.ds(0, _EBLK)], dbufb, sembd).wait()
                pltpu.make_async_copy(
                    src_hbm.at[pl.ds(0, _EBLK)], sbufb, sembs).wait()

                @pl.when(b0 + 2 < _NBLK)
                def _():
                    pltpu.async_copy(
                        dst_hbm.at[pl.ds((b0 + 2) * _EBLK, _EBLK)],
                        dbufa, semad)
                    pltpu.async_copy(
                        src_hbm.at[pl.ds((b0 + 2) * _EBLK, _EBLK)],
                        sbufa, semas)

                return scan_block(dbufb, sbufb, off)

            off = lax.fori_loop(0, _NBLK // 2, blk2_body, jnp.int32(0))

            @pl.when(off > 0)
            def _(off=off, lo=lo):
                offv = jnp.full((_VL,), off, jnp.int32)
                dumpv = jnp.full((_VL,), lo + _CH, jnp.int32)
                dv = jnp.where(li < offv, std[pl.ds(0, _VL)], dumpv)
                sv = jnp.where(li < offv, sts[pl.ds(0, _VL)],
                               jnp.zeros((_VL,), jnp.int32))
                drain16(dv, sv, lo)

            pltpu.sync_copy(acc.at[pl.ds(0, _CH)], out_hbm.at[g])

    return kern(xl, xr_pad, srcv, dstv, attb, zrows)


def kernel(x, edge_index, batch, params):
    p = params
    srcv = edge_index[0]
    dstv = edge_index[1]
    batch2d = batch.reshape(1, _N)
    zrows = jnp.zeros((_ACC_ROWS, _ACW), jnp.float32)

    h = _proj(x, p["proj_W"], p["proj_b"].reshape(1, _HID))
    for i in range(3):
        att = p[f"att{i}"]
        attb = att.reshape(_HD)
        a_bd = jnp.zeros((_HD, _HEADS), jnp.float32)
        for hh in range(_HEADS):
            a_bd = a_bd.at[hh * _HID:(hh + 1) * _HID, hh].set(att[hh])

        xl, xr = _mm2(h, p[f"Wl{i}"], p[f"bl{i}"].reshape(1, _HD),
                      p[f"Wr{i}"], p[f"br{i}"].reshape(1, _HD))
        xr_pad = jnp.concatenate(
            [xr, jnp.zeros((_XR_PAD_ROWS - _N, _HD), jnp.float32)], axis=0)

        if _DEBUG_JNP_EDGE:
            keep = srcv != dstv
            d0 = jnp.where(keep, dstv, _N)
            zse = xl[srcv] + xr[jnp.minimum(d0, _N - 1)]
            tse = jnp.maximum(zse, 0.2 * zse)
            al = (tse.reshape(_E, _HEADS, _HID)
                  * att[None, :, :]).sum(-1)
            exe = jnp.exp(al)
            scd8 = jax.ops.segment_sum(exe, d0, num_segments=_N)
            scm = jnp.concatenate([
                jax.ops.segment_sum(
                    xl[srcv, hh * _HID:(hh + 1) * _HID] * exe[:, hh:hh + 1],
                    d0, num_segments=_N)
                for hh in range(_HEADS)], axis=1)
            scd = jnp.concatenate(
                [scd8, jnp.zeros((_N, 8), jnp.float32)], axis=1)
        else:
            scg = _sc_edge(xl, xr_pad, srcv, dstv, attb, zrows)
            scflat = scg.reshape(_NGROUP * _CH, _ACW)
            scm = scflat[:_N, :_HD]
            scd = scflat[:_N, _HD:]
        sem, sed = _selfloop(xl, xr, a_bd)

        c = _HD if _CONCAT[i] else _HID
        o = _epi_a(scm, scd, sem, sed, p[f"bias{i}"].reshape(1, c),
                   _CONCAT[i])
        h = _epi_b(o, p[f"bn{i}_g"].reshape(1, c),
                   p[f"bn{i}_b"].reshape(1, c), do_relu=(i < 2))

    return _head(h, batch2d, p["out_W1"], p["out_b1"].reshape(1, _HID // 2),
                 p["out_bn_g"].reshape(1, _HID // 2),
                 p["out_bn_b"].reshape(1, _HID // 2),
                 p["out_W2"], p["out_b2"].reshape(1, 1))
